# baseline (speedup 1.0000x reference)
import dataclasses
import functools

import numpy as np

import jax
import jax.numpy as jnp
from jax import lax
from jax.experimental import pallas as pl
from jax.experimental.pallas import tpu as pltpu
from jax.experimental.pallas import tpu_sc as plsc

N_NODES = 10000
N_EDGES = 320000
D = 128
ALPHA = 0.2

NC = 2
NS = 16
L = 16
NW = NC * NS

B = 80
EDGES_PER_W = N_EDGES // NW
NBLK = EDGES_PER_W // B
GROUPS = B // L

ZCH = 80
NZCH = N_NODES // ZCH
RCH = 400
NRCH = N_NODES // RCH

_mesh = plsc.VectorSubcoreMesh(core_axis_name="c", subcore_axis_name="s")

_cp = pltpu.CompilerParams()
if "needs_layout_passes" in pltpu.CompilerParams.__dataclass_fields__:
    _cp = dataclasses.replace(_cp, needs_layout_passes=False)

_cp_notc = dataclasses.replace(_cp, use_tc_tiling_on_sc=False)

_PERM = np.empty((D,), np.int32)
for _k in range(D // 32):
    for _i in range(16):
        _PERM[32 * _k + 2 * _i] = 32 * _k + _i
        _PERM[32 * _k + 2 * _i + 1] = 32 * _k + 16 + _i


def _dot16(ri_v, rj_v, e):
    a = ri_v[e, pl.ds(0, L)] * rj_v[e, pl.ds(0, L)]
    for c in range(1, D // L):
        a = a + ri_v[e, pl.ds(c * L, L)] * rj_v[e, pl.ds(c * L, L)]
    return a


@functools.partial(
    pl.kernel,
    out_type=[
        jax.ShapeDtypeStruct((N_EDGES,), jnp.float32),
        jax.ShapeDtypeStruct((NW, 2, L), jnp.float32),
    ],
    mesh=_mesh,
    scratch_types=[
        pltpu.VMEM((B, D), jnp.float32),
        pltpu.VMEM((B, D), jnp.float32),
        pltpu.VMEM((B, D), jnp.float32),
        pltpu.VMEM((B, D), jnp.float32),
        pltpu.VMEM((2, B), jnp.int32),
        pltpu.VMEM((2, B), jnp.int32),
        pltpu.VMEM((2, B), jnp.float32),
        pltpu.VMEM((L, L), jnp.float32),
        pltpu.VMEM((L, L), jnp.float32),
        pltpu.VMEM((2, L), jnp.float32),
        pltpu.VMEM_SHARED((N_NODES, D), jnp.float32),
        pltpu.SemaphoreType.DMA,
        pltpu.SemaphoreType.DMA,
        pltpu.SemaphoreType.DMA,
        pltpu.SemaphoreType.DMA,
        pltpu.SemaphoreType.DMA,
        pltpu.SemaphoreType.DMA,
    ],
    compiler_params=_cp,
)
def _score_kernel(emb_hbm, src2_hbm, dst2_hbm, scores_hbm, stats_hbm,
                  ri0, ri1, rj0, rj1, src_ix, dst_ix, s_vv, red0, red1, mz_v,
                  tab_sh, si0, si1, sr0, sr1, ss0, ss1):
    sid = lax.axis_index("s")
    wid = lax.axis_index("c") * NS + sid
    base = wid * EDGES_PER_W
    wblk = wid * NBLK
    lanes = lax.iota(jnp.int32, L)
    last = jnp.full((L,), L - 1, jnp.int32)
    ri = (ri0, ri1)
    rj = (rj0, rj1)
    semi = (si0, si1)
    semr = (sr0, sr1)
    sems = (ss0, ss1)

    @pl.loop(0, NZCH // NS)
    def _tc(i):
        r0 = (i * NS + sid) * ZCH
        pltpu.sync_copy(emb_hbm.at[pl.ds(r0, ZCH)], tab_sh.at[pl.ds(r0, ZCH)])
    _trem = NZCH % NS
    if _trem:
        @pl.when(sid < _trem)
        def _():
            r0 = ((NZCH - _trem) + sid) * ZCH
            pltpu.sync_copy(emb_hbm.at[pl.ds(r0, ZCH)],
                            tab_sh.at[pl.ds(r0, ZCH)])
    plsc.subcore_barrier()

    mz_v[0, :] = jnp.full((L,), -1e30, jnp.float32)
    mz_v[1, :] = jnp.zeros((L,), jnp.float32)

    def idx_copies(slot, b):
        return (pltpu.make_async_copy(src2_hbm.at[wblk + b],
                                      src_ix.at[slot], semi[slot]),
                pltpu.make_async_copy(dst2_hbm.at[wblk + b],
                                      dst_ix.at[slot], semi[slot]))

    def row_copies(slot):
        return (pltpu.make_async_copy(tab_sh.at[src_ix.at[slot]],
                                      ri[slot], semr[slot]),
                pltpu.make_async_copy(tab_sh.at[dst_ix.at[slot]],
                                      rj[slot], semr[slot]))

    def score_copy(slot, b):
        return pltpu.make_async_copy(
            s_vv.at[slot], scores_hbm.at[pl.ds(base + b * B, B)], sems[slot])

    def compute(slot, b):
        @pl.loop(0, GROUPS)
        def _grp(g):
            for e0 in range(0, L, 8):
                accs = [ri[slot][g * L + e0 + k, pl.ds(0, L)]
                        * rj[slot][g * L + e0 + k, pl.ds(0, L)]
                        for k in range(8)]
                for c in range(1, D // L):
                    for k in range(8):
                        accs[k] = accs[k] + (
                            ri[slot][g * L + e0 + k, pl.ds(c * L, L)]
                            * rj[slot][g * L + e0 + k, pl.ds(c * L, L)])
                for k in range(8):
                    red0[e0 + k, :] = accs[k]
            s16 = plsc.load_gather(red0, [lanes, jnp.full((L,), 0, jnp.int32)])
            for l in range(1, L):
                s16 = s16 + plsc.load_gather(
                    red0, [lanes, jnp.full((L,), l, jnp.int32)])
            s16 = jnp.where(s16 > 0.0, s16, ALPHA * s16)
            s_vv[slot, pl.ds(g * L, L)] = s16
            m_old = mz_v[0, :]
            m_new = jnp.maximum(m_old, s16)
            mz_v[1, :] = (mz_v[1, :] * jnp.exp(m_old - m_new)
                          + jnp.exp(s16 - m_new))
            mz_v[0, :] = m_new
        score_copy(slot, b).start()

    for c in idx_copies(0, 0):
        c.start()
    for c in idx_copies(0, 0):
        c.wait()
    for c in row_copies(0):
        c.start()
    for c in idx_copies(1, 1):
        c.start()

    @pl.loop(0, NBLK // 2)
    def _pipe(i):
        for par in range(2):
            b = 2 * i + par
            slot, other = par, 1 - par
            for c in row_copies(slot):
                c.wait()

            @pl.when(b < NBLK - 2)
            def _():
                for c in idx_copies(slot, b + 2):
                    c.start()
            for c in idx_copies(other, b + 1):
                c.wait()
            for c in row_copies(other):
                c.start()

            @pl.when(b >= 2)
            def _():
                score_copy(slot, b - 2).wait()
            compute(slot, b)

    bl = NBLK - 1
    for c in row_copies(bl % 2):
        c.wait()
    score_copy(bl % 2, bl - 2).wait()
    compute(bl % 2, bl)
    score_copy(1 - bl % 2, bl - 1).wait()
    score_copy(bl % 2, bl).wait()

    m_acc = mz_v[0, :]
    m_w = jnp.max(m_acc)
    mv = jnp.full((L,), m_w, jnp.float32)
    z_w = jnp.sum(mz_v[1, :] * jnp.exp(m_acc - mv))
    mz_v[0, :] = mv
    mz_v[1, :] = jnp.full((L,), z_w, jnp.float32)
    pltpu.sync_copy(mz_v, stats_hbm.at[wid])


@functools.partial(
    pl.kernel,
    out_type=jax.ShapeDtypeStruct((NC, N_NODES, D), jnp.float32),
    mesh=_mesh,
    scratch_types=[
        pltpu.VMEM((B, D), jnp.float32),
        pltpu.VMEM((B, D), jnp.float32),
        pltpu.VMEM((B, D), jnp.float32),
        pltpu.VMEM((6, B), jnp.int32),
        pltpu.VMEM((6, B), jnp.int32),
        pltpu.VMEM((2, B), jnp.float32),
        pltpu.VMEM((NW, 2, L), jnp.float32),
        pltpu.VMEM_SHARED((N_NODES, D), jnp.float32),
        pltpu.SemaphoreType.DMA,
        pltpu.SemaphoreType.DMA,
        pltpu.SemaphoreType.DMA,
        pltpu.SemaphoreType.DMA,
        pltpu.SemaphoreType.DMA,
        pltpu.SemaphoreType.DMA,
        pltpu.SemaphoreType.DMA,
        pltpu.SemaphoreType.DMA,
        pltpu.SemaphoreType.DMA,
        pltpu.SemaphoreType.DMA,
        pltpu.SemaphoreType.DMA,
        pltpu.SemaphoreType.DMA,
        pltpu.SemaphoreType.DMA,
        pltpu.SemaphoreType.DMA,
    ],
    compiler_params=_cp,
)
def _agg_kernel(emb_hbm, src2_hbm, dst2_hbm, scores_hbm, stats_hbm, part_hbm,
                rb0, rb1, rb2, src_ix, dst_ix, s_vv, st_v, acc_sh,
                bi0, bi1, bi2, bi3, bi4, bi5, br0, br1, br2,
                bw0, bw1, bw2, bs0, bs1):
    cid = lax.axis_index("c")
    sid = lax.axis_index("s")
    wid = cid * NS + sid
    base = wid * EDGES_PER_W
    wblk = wid * NBLK
    rb = (rb0, rb1, rb2)
    semi = (bi0, bi1, bi2, bi3, bi4, bi5)
    semr = (br0, br1, br2)
    semw = (bw0, bw1, bw2)
    sems = (bs0, bs1)

    pltpu.sync_copy(stats_hbm, st_v)
    m_v = st_v[0, 0, :]
    for w in range(1, NW):
        m_v = jnp.maximum(m_v, st_v[w, 0, :])
    z_all = st_v[0, 1, :] * jnp.exp(st_v[0, 0, :] - m_v)
    for w in range(1, NW):
        z_all = z_all + st_v[w, 1, :] * jnp.exp(st_v[w, 0, :] - m_v)
    inv_z = 1.0 / z_all

    @pl.loop(0, ZCH)
    def _zr(r):
        for c in range(D // L):
            rb0[r, pl.ds(c * L, L)] = jnp.zeros((L,), jnp.float32)

    @pl.loop(0, NZCH // NS)
    def _zc(i):
        pltpu.sync_copy(rb0, acc_sh.at[pl.ds((i * NS + sid) * ZCH, ZCH)])
    rem = NZCH % NS
    if rem:
        @pl.when(sid < rem)
        def _():
            pltpu.sync_copy(
                rb0, acc_sh.at[pl.ds(((NZCH - rem) + sid) * ZCH, ZCH)])
    plsc.subcore_barrier()

    def idx_copies(islot, b):
        return (pltpu.make_async_copy(src2_hbm.at[wblk + b],
                                      src_ix.at[islot], semi[islot]),
                pltpu.make_async_copy(dst2_hbm.at[wblk + b],
                                      dst_ix.at[islot], semi[islot]))

    def row_copy(rslot, islot):
        return pltpu.make_async_copy(emb_hbm.at[dst_ix.at[islot]],
                                     rb[rslot], semr[rslot])

    def score_copy(rslot, b):
        return pltpu.make_async_copy(
            scores_hbm.at[pl.ds(base + b * B, B)], s_vv.at[rslot], sems[rslot])

    class _Scatter:
        def __init__(self, rslot, islot):
            self._d = pltpu.make_async_copy(
                rb[rslot], acc_sh.at[src_ix.at[islot]], semw[rslot])

        def start(self):
            self._d.start(add=True)

        def wait(self):
            self._d.wait()

    def scatter_copy(rslot, islot):
        return _Scatter(rslot, islot)

    def compute(rslot, sslot):
        slot_v = jnp.full((L,), sslot, jnp.int32)

        @pl.loop(0, B)
        def _edge(e):
            srow = plsc.load_gather(s_vv, [slot_v, jnp.full((L,), e, jnp.int32)])
            wrow = jnp.exp(srow - m_v) * inv_z
            for c in range(D // L):
                rb[rslot][e, pl.ds(c * L, L)] = (
                    rb[rslot][e, pl.ds(c * L, L)] * wrow)

    for c in idx_copies(0, 0):
        c.start()
    for c in idx_copies(0, 0):
        c.wait()
    row_copy(0, 0).start()
    score_copy(0, 0).start()
    for c in idx_copies(1, 1):
        c.start()
    score_copy(1, 1).start()

    @pl.loop(0, (NBLK + 5) // 6)
    def _pipe(i):
        for par in range(6):
            b = 6 * i + par
            r3, r3n = par % 3, (par + 1) % 3
            s2 = par % 2

            @pl.when(b < NBLK)
            def _():
                row_copy(r3, par).wait()

            @pl.when((b >= 2) & (b < NBLK + 2))
            def _():
                scatter_copy((par + 1) % 3, (par + 4) % 6).wait()

            @pl.when(b < NBLK - 2)
            def _():
                for c in idx_copies((par + 2) % 6, b + 2):
                    c.start()

            @pl.when(b < NBLK - 1)
            def _():
                for c in idx_copies((par + 1) % 6, b + 1):
                    c.wait()
                row_copy(r3n, (par + 1) % 6).start()

            @pl.when(b < NBLK)
            def _():
                score_copy(s2, b).wait()
                compute(r3, s2)
                scatter_copy(r3, par).start()

            @pl.when(b < NBLK - 2)
            def _():
                score_copy(s2, b + 2).start()

    scatter_copy((NBLK - 1) % 3, (NBLK - 1) % 6).wait()
    plsc.subcore_barrier()

    for ch in range(NRCH):
        @pl.when(sid == (ch % NS))
        def _():
            pltpu.sync_copy(acc_sh.at[pl.ds(ch * RCH, RCH)],
                            part_hbm.at[cid, pl.ds(ch * RCH, RCH)])


def _final_body(e_ref, p0_ref, p1_ref, o_ref):
    o_ref[...] = e_ref[...] + p0_ref[...] + p1_ref[...]


def _final_add(emb, p0, p1):
    blk = 1000
    grid = N_NODES // blk
    spec = pl.BlockSpec((blk, D), lambda i: (i, 0))
    return pl.pallas_call(
        _final_body,
        out_shape=jax.ShapeDtypeStruct((N_NODES, D), jnp.float32),
        grid=(grid,),
        in_specs=[spec, spec, spec],
        out_specs=spec,
    )(emb, p0, p1)


def kernel(drug_embeddings, drug_relationships):
    src = drug_relationships[:, 0].astype(jnp.int32)
    dst = drug_relationships[:, 1].astype(jnp.int32)
    src2 = src.reshape(NW * NBLK, B)
    dst2 = dst.reshape(NW * NBLK, B)
    scores, stats = _score_kernel(drug_embeddings, src2, dst2)
    partial = _agg_kernel(drug_embeddings, src2, dst2, scores, stats)
    return _final_add(drug_embeddings, partial[0], partial[1])


# speedup vs baseline: 1.1549x; 1.1549x over previous
import dataclasses
import functools

import numpy as np

import jax
import jax.numpy as jnp
from jax import lax
from jax.experimental import pallas as pl
from jax.experimental.pallas import tpu as pltpu
from jax.experimental.pallas import tpu_sc as plsc

N_NODES = 10000
N_EDGES = 320000
D = 128
ALPHA = 0.2

NC = 2
NS = 16
L = 16
NW = NC * NS

B = 80
EDGES_PER_W = N_EDGES // NW
NBLK = EDGES_PER_W // B
GROUPS = B // L

ZCH = 80
NZCH = N_NODES // ZCH
RCH = 400
NRCH = N_NODES // RCH

_mesh = plsc.VectorSubcoreMesh(core_axis_name="c", subcore_axis_name="s")

_cp = pltpu.CompilerParams()
if "needs_layout_passes" in pltpu.CompilerParams.__dataclass_fields__:
    _cp = dataclasses.replace(_cp, needs_layout_passes=False)

_cp_notc = dataclasses.replace(_cp, use_tc_tiling_on_sc=False)

_PERM = np.empty((D,), np.int32)
for _k in range(D // 32):
    for _i in range(16):
        _PERM[32 * _k + 2 * _i] = 32 * _k + _i
        _PERM[32 * _k + 2 * _i + 1] = 32 * _k + 16 + _i


def _dot16(ri_v, rj_v, e):
    a = ri_v[e, pl.ds(0, L)] * rj_v[e, pl.ds(0, L)]
    for c in range(1, D // L):
        a = a + ri_v[e, pl.ds(c * L, L)] * rj_v[e, pl.ds(c * L, L)]
    return a


@functools.partial(
    pl.kernel,
    out_type=[
        jax.ShapeDtypeStruct((N_EDGES,), jnp.float32),
        jax.ShapeDtypeStruct((NW, 2, L), jnp.float32),
    ],
    mesh=_mesh,
    scratch_types=[
        pltpu.VMEM((B, D), jnp.float32),
        pltpu.VMEM((B, D), jnp.float32),
        pltpu.VMEM((B, D), jnp.float32),
        pltpu.VMEM((B, D), jnp.float32),
        pltpu.VMEM((2, B), jnp.int32),
        pltpu.VMEM((2, B), jnp.int32),
        pltpu.VMEM((2, B), jnp.float32),
        pltpu.VMEM((L, L), jnp.float32),
        pltpu.VMEM((L, L), jnp.float32),
        pltpu.VMEM((2, L), jnp.float32),
        pltpu.VMEM_SHARED((N_NODES, D), jnp.float32),
        pltpu.SemaphoreType.DMA,
        pltpu.SemaphoreType.DMA,
        pltpu.SemaphoreType.DMA,
        pltpu.SemaphoreType.DMA,
        pltpu.SemaphoreType.DMA,
        pltpu.SemaphoreType.DMA,
    ],
    compiler_params=_cp,
)
def _score_kernel(emb_hbm, src2_hbm, dst2_hbm, scores_hbm, stats_hbm,
                  ri0, ri1, rj0, rj1, src_ix, dst_ix, s_vv, red0, red1, mz_v,
                  tab_sh, si0, si1, sr0, sr1, ss0, ss1):
    sid = lax.axis_index("s")
    wid = lax.axis_index("c") * NS + sid
    base = wid * EDGES_PER_W
    wblk = wid * NBLK
    lanes = lax.iota(jnp.int32, L)
    last = jnp.full((L,), L - 1, jnp.int32)
    ri = (ri0, ri1)
    rj = (rj0, rj1)
    semi = (si0, si1)
    semr = (sr0, sr1)
    sems = (ss0, ss1)

    @pl.loop(0, NZCH // NS)
    def _tc(i):
        r0 = (i * NS + sid) * ZCH
        pltpu.sync_copy(emb_hbm.at[pl.ds(r0, ZCH)], tab_sh.at[pl.ds(r0, ZCH)])
    _trem = NZCH % NS
    if _trem:
        @pl.when(sid < _trem)
        def _():
            r0 = ((NZCH - _trem) + sid) * ZCH
            pltpu.sync_copy(emb_hbm.at[pl.ds(r0, ZCH)],
                            tab_sh.at[pl.ds(r0, ZCH)])
    plsc.subcore_barrier()

    mz_v[0, :] = jnp.full((L,), -1e30, jnp.float32)
    mz_v[1, :] = jnp.zeros((L,), jnp.float32)

    def idx_copies(slot, b):
        return (pltpu.make_async_copy(src2_hbm.at[wblk + b],
                                      src_ix.at[slot], semi[slot]),
                pltpu.make_async_copy(dst2_hbm.at[wblk + b],
                                      dst_ix.at[slot], semi[slot]))

    def row_copies(slot):
        return (pltpu.make_async_copy(tab_sh.at[src_ix.at[slot]],
                                      ri[slot], semr[slot]),
                pltpu.make_async_copy(tab_sh.at[dst_ix.at[slot]],
                                      rj[slot], semr[slot]))

    def score_copy(slot, b):
        return pltpu.make_async_copy(
            s_vv.at[slot], scores_hbm.at[pl.ds(base + b * B, B)], sems[slot])

    def compute(slot, b):
        @pl.loop(0, GROUPS)
        def _grp(g):
            for e0 in range(0, L, 4):
                accs = [ri[slot][g * L + e0 + k, pl.ds(0, L)]
                        * rj[slot][g * L + e0 + k, pl.ds(0, L)]
                        for k in range(4)]
                for c in range(1, D // L):
                    for k in range(4):
                        accs[k] = accs[k] + (
                            ri[slot][g * L + e0 + k, pl.ds(c * L, L)]
                            * rj[slot][g * L + e0 + k, pl.ds(c * L, L)])
                for k in range(4):
                    red0[e0 + k, :] = accs[k]
            s16 = plsc.load_gather(red0, [lanes, jnp.full((L,), 0, jnp.int32)])
            for l in range(1, L):
                s16 = s16 + plsc.load_gather(
                    red0, [lanes, jnp.full((L,), l, jnp.int32)])
            s16 = jnp.where(s16 > 0.0, s16, ALPHA * s16)
            s_vv[slot, pl.ds(g * L, L)] = s16
            m_old = mz_v[0, :]
            m_new = jnp.maximum(m_old, s16)
            mz_v[1, :] = (mz_v[1, :] * jnp.exp(m_old - m_new)
                          + jnp.exp(s16 - m_new))
            mz_v[0, :] = m_new
        score_copy(slot, b).start()

    for c in idx_copies(0, 0):
        c.start()
    for c in idx_copies(0, 0):
        c.wait()
    for c in row_copies(0):
        c.start()
    for c in idx_copies(1, 1):
        c.start()

    @pl.loop(0, NBLK // 2)
    def _pipe(i):
        for par in range(2):
            b = 2 * i + par
            slot, other = par, 1 - par
            for c in row_copies(slot):
                c.wait()

            @pl.when(b < NBLK - 2)
            def _():
                for c in idx_copies(slot, b + 2):
                    c.start()
            for c in idx_copies(other, b + 1):
                c.wait()
            for c in row_copies(other):
                c.start()

            @pl.when(b >= 2)
            def _():
                score_copy(slot, b - 2).wait()
            compute(slot, b)

    bl = NBLK - 1
    for c in row_copies(bl % 2):
        c.wait()
    score_copy(bl % 2, bl - 2).wait()
    compute(bl % 2, bl)
    score_copy(1 - bl % 2, bl - 1).wait()
    score_copy(bl % 2, bl).wait()

    m_acc = mz_v[0, :]
    m_w = jnp.max(m_acc)
    mv = jnp.full((L,), m_w, jnp.float32)
    z_w = jnp.sum(mz_v[1, :] * jnp.exp(m_acc - mv))
    mz_v[0, :] = mv
    mz_v[1, :] = jnp.full((L,), z_w, jnp.float32)
    pltpu.sync_copy(mz_v, stats_hbm.at[wid])


@functools.partial(
    pl.kernel,
    out_type=jax.ShapeDtypeStruct((NC, N_NODES, D), jnp.float32),
    mesh=_mesh,
    scratch_types=[
        pltpu.VMEM((B, D), jnp.float32),
        pltpu.VMEM((B, D), jnp.float32),
        pltpu.VMEM((B, D), jnp.float32),
        pltpu.VMEM((6, B), jnp.int32),
        pltpu.VMEM((6, B), jnp.int32),
        pltpu.VMEM((2, B), jnp.float32),
        pltpu.VMEM((NW, 2, L), jnp.float32),
        pltpu.VMEM_SHARED((N_NODES, D), jnp.float32),
        pltpu.SemaphoreType.DMA,
        pltpu.SemaphoreType.DMA,
        pltpu.SemaphoreType.DMA,
        pltpu.SemaphoreType.DMA,
        pltpu.SemaphoreType.DMA,
        pltpu.SemaphoreType.DMA,
        pltpu.SemaphoreType.DMA,
        pltpu.SemaphoreType.DMA,
        pltpu.SemaphoreType.DMA,
        pltpu.SemaphoreType.DMA,
        pltpu.SemaphoreType.DMA,
        pltpu.SemaphoreType.DMA,
        pltpu.SemaphoreType.DMA,
        pltpu.SemaphoreType.DMA,
    ],
    compiler_params=_cp,
)
def _agg_kernel(emb_hbm, src2_hbm, dst2_hbm, scores_hbm, stats_hbm, part_hbm,
                rb0, rb1, rb2, src_ix, dst_ix, s_vv, st_v, acc_sh,
                bi0, bi1, bi2, bi3, bi4, bi5, br0, br1, br2,
                bw0, bw1, bw2, bs0, bs1):
    cid = lax.axis_index("c")
    sid = lax.axis_index("s")
    wid = cid * NS + sid
    base = wid * EDGES_PER_W
    wblk = wid * NBLK
    rb = (rb0, rb1, rb2)
    semi = (bi0, bi1, bi2, bi3, bi4, bi5)
    semr = (br0, br1, br2)
    semw = (bw0, bw1, bw2)
    sems = (bs0, bs1)

    pltpu.sync_copy(stats_hbm, st_v)
    m_v = st_v[0, 0, :]
    for w in range(1, NW):
        m_v = jnp.maximum(m_v, st_v[w, 0, :])
    z_all = st_v[0, 1, :] * jnp.exp(st_v[0, 0, :] - m_v)
    for w in range(1, NW):
        z_all = z_all + st_v[w, 1, :] * jnp.exp(st_v[w, 0, :] - m_v)
    inv_z = 1.0 / z_all

    @pl.loop(0, ZCH)
    def _zr(r):
        for c in range(D // L):
            rb0[r, pl.ds(c * L, L)] = jnp.zeros((L,), jnp.float32)

    @pl.loop(0, NZCH // NS)
    def _zc(i):
        pltpu.sync_copy(rb0, acc_sh.at[pl.ds((i * NS + sid) * ZCH, ZCH)])
    rem = NZCH % NS
    if rem:
        @pl.when(sid < rem)
        def _():
            pltpu.sync_copy(
                rb0, acc_sh.at[pl.ds(((NZCH - rem) + sid) * ZCH, ZCH)])
    plsc.subcore_barrier()

    def idx_copies(islot, b):
        return (pltpu.make_async_copy(src2_hbm.at[wblk + b],
                                      src_ix.at[islot], semi[islot]),
                pltpu.make_async_copy(dst2_hbm.at[wblk + b],
                                      dst_ix.at[islot], semi[islot]))

    def row_copy(rslot, islot):
        return pltpu.make_async_copy(emb_hbm.at[dst_ix.at[islot]],
                                     rb[rslot], semr[rslot])

    def score_copy(rslot, b):
        return pltpu.make_async_copy(
            scores_hbm.at[pl.ds(base + b * B, B)], s_vv.at[rslot], sems[rslot])

    class _Scatter:
        def __init__(self, rslot, islot):
            self._d = pltpu.make_async_copy(
                rb[rslot], acc_sh.at[src_ix.at[islot]], semw[rslot])

        def start(self):
            self._d.start(add=True)

        def wait(self):
            self._d.wait()

    def scatter_copy(rslot, islot):
        return _Scatter(rslot, islot)

    def compute(rslot, sslot):
        slot_v = jnp.full((L,), sslot, jnp.int32)

        @pl.loop(0, B, step=4)
        def _edge(e):
            wrows = []
            for k in range(4):
                srow = plsc.load_gather(
                    s_vv, [slot_v, jnp.full((L,), e + k, jnp.int32)])
                wrows.append(jnp.exp(srow - m_v) * inv_z)
            for c in range(D // L):
                for k in range(4):
                    rb[rslot][e + k, pl.ds(c * L, L)] = (
                        rb[rslot][e + k, pl.ds(c * L, L)] * wrows[k])

    for c in idx_copies(0, 0):
        c.start()
    for c in idx_copies(0, 0):
        c.wait()
    row_copy(0, 0).start()
    score_copy(0, 0).start()
    for c in idx_copies(1, 1):
        c.start()
    score_copy(1, 1).start()

    @pl.loop(0, (NBLK + 5) // 6)
    def _pipe(i):
        for par in range(6):
            b = 6 * i + par
            r3, r3n = par % 3, (par + 1) % 3
            s2 = par % 2

            @pl.when(b < NBLK)
            def _():
                row_copy(r3, par).wait()

            @pl.when((b >= 2) & (b < NBLK + 2))
            def _():
                scatter_copy((par + 1) % 3, (par + 4) % 6).wait()

            @pl.when(b < NBLK - 2)
            def _():
                for c in idx_copies((par + 2) % 6, b + 2):
                    c.start()

            @pl.when(b < NBLK - 1)
            def _():
                for c in idx_copies((par + 1) % 6, b + 1):
                    c.wait()
                row_copy(r3n, (par + 1) % 6).start()

            @pl.when(b < NBLK)
            def _():
                score_copy(s2, b).wait()
                compute(r3, s2)
                scatter_copy(r3, par).start()

            @pl.when(b < NBLK - 2)
            def _():
                score_copy(s2, b + 2).start()

    scatter_copy((NBLK - 1) % 3, (NBLK - 1) % 6).wait()
    plsc.subcore_barrier()

    for ch in range(NRCH):
        @pl.when(sid == (ch % NS))
        def _():
            pltpu.sync_copy(acc_sh.at[pl.ds(ch * RCH, RCH)],
                            part_hbm.at[cid, pl.ds(ch * RCH, RCH)])


def _final_body(e_ref, p0_ref, p1_ref, o_ref):
    o_ref[...] = e_ref[...] + p0_ref[...] + p1_ref[...]


def _final_add(emb, p0, p1):
    blk = 1000
    grid = N_NODES // blk
    spec = pl.BlockSpec((blk, D), lambda i: (i, 0))
    return pl.pallas_call(
        _final_body,
        out_shape=jax.ShapeDtypeStruct((N_NODES, D), jnp.float32),
        grid=(grid,),
        in_specs=[spec, spec, spec],
        out_specs=spec,
    )(emb, p0, p1)


def kernel(drug_embeddings, drug_relationships):
    src = drug_relationships[:, 0].astype(jnp.int32)
    dst = drug_relationships[:, 1].astype(jnp.int32)
    src2 = src.reshape(NW * NBLK, B)
    dst2 = dst.reshape(NW * NBLK, B)
    scores, stats = _score_kernel(drug_embeddings, src2, dst2)
    partial = _agg_kernel(drug_embeddings, src2, dst2, scores, stats)
    return _final_add(drug_embeddings, partial[0], partial[1])


# speedup vs baseline: 1.1565x; 1.0014x over previous
import dataclasses
import functools

import jax
import jax.numpy as jnp
from jax import lax
from jax.experimental import pallas as pl
from jax.experimental.pallas import tpu as pltpu
from jax.experimental.pallas import tpu_sc as plsc

N_NODES = 10000
N_EDGES = 320000
D = 128
ALPHA = 0.2

NC = 2
NS = 16
L = 16
NW = NC * NS

B = 80
EDGES_PER_W = N_EDGES // NW
NBLK = EDGES_PER_W // B
GROUPS = B // L

ZCH = 80
NZCH = N_NODES // ZCH
RCH = 400
NRCH = N_NODES // RCH

_mesh = plsc.VectorSubcoreMesh(core_axis_name="c", subcore_axis_name="s")

_cp = pltpu.CompilerParams()
if "needs_layout_passes" in pltpu.CompilerParams.__dataclass_fields__:
    _cp = dataclasses.replace(_cp, needs_layout_passes=False)

@functools.partial(
    pl.kernel,
    out_type=[
        jax.ShapeDtypeStruct((N_EDGES,), jnp.float32),
        jax.ShapeDtypeStruct((NW, 2, L), jnp.float32),
    ],
    mesh=_mesh,
    scratch_types=[
        pltpu.VMEM((B, D), jnp.float32),
        pltpu.VMEM((B, D), jnp.float32),
        pltpu.VMEM((B, D), jnp.float32),
        pltpu.VMEM((B, D), jnp.float32),
        pltpu.VMEM((2, B), jnp.int32),
        pltpu.VMEM((2, B), jnp.int32),
        pltpu.VMEM((2, B), jnp.float32),
        pltpu.VMEM((L, L), jnp.float32),
        pltpu.VMEM((L, L), jnp.float32),
        pltpu.VMEM((2, L), jnp.float32),
        pltpu.VMEM_SHARED((N_NODES, D), jnp.float32),
        pltpu.SemaphoreType.DMA,
        pltpu.SemaphoreType.DMA,
        pltpu.SemaphoreType.DMA,
        pltpu.SemaphoreType.DMA,
        pltpu.SemaphoreType.DMA,
        pltpu.SemaphoreType.DMA,
    ],
    compiler_params=_cp,
)
def _score_kernel(emb_hbm, src2_hbm, dst2_hbm, scores_hbm, stats_hbm,
                  ri0, ri1, rj0, rj1, src_ix, dst_ix, s_vv, red0, red1, mz_v,
                  tab_sh, si0, si1, sr0, sr1, ss0, ss1):
    sid = lax.axis_index("s")
    wid = lax.axis_index("c") * NS + sid
    base = wid * EDGES_PER_W
    wblk = wid * NBLK
    lanes = lax.iota(jnp.int32, L)
    ri = (ri0, ri1)
    rj = (rj0, rj1)
    semi = (si0, si1)
    semr = (sr0, sr1)
    sems = (ss0, ss1)

    @pl.loop(0, NZCH // NS)
    def _tc(i):
        r0 = (i * NS + sid) * ZCH
        pltpu.sync_copy(emb_hbm.at[pl.ds(r0, ZCH)], tab_sh.at[pl.ds(r0, ZCH)])
    _trem = NZCH % NS
    if _trem:
        @pl.when(sid < _trem)
        def _():
            r0 = ((NZCH - _trem) + sid) * ZCH
            pltpu.sync_copy(emb_hbm.at[pl.ds(r0, ZCH)],
                            tab_sh.at[pl.ds(r0, ZCH)])
    plsc.subcore_barrier()

    mz_v[0, :] = jnp.full((L,), -1e30, jnp.float32)
    mz_v[1, :] = jnp.zeros((L,), jnp.float32)

    def idx_copies(slot, b):
        return (pltpu.make_async_copy(src2_hbm.at[wblk + b],
                                      src_ix.at[slot], semi[slot]),
                pltpu.make_async_copy(dst2_hbm.at[wblk + b],
                                      dst_ix.at[slot], semi[slot]))

    def row_copies(slot):
        return (pltpu.make_async_copy(tab_sh.at[src_ix.at[slot]],
                                      ri[slot], semr[slot]),
                pltpu.make_async_copy(tab_sh.at[dst_ix.at[slot]],
                                      rj[slot], semr[slot]))

    def score_copy(slot, b):
        return pltpu.make_async_copy(
            s_vv.at[slot], scores_hbm.at[pl.ds(base + b * B, B)], sems[slot])

    def compute(slot, b):
        @pl.loop(0, GROUPS)
        def _grp(g):
            for e0 in range(0, L, 4):
                accs = [ri[slot][g * L + e0 + k, pl.ds(0, L)]
                        * rj[slot][g * L + e0 + k, pl.ds(0, L)]
                        for k in range(4)]
                for c in range(1, D // L):
                    for k in range(4):
                        accs[k] = accs[k] + (
                            ri[slot][g * L + e0 + k, pl.ds(c * L, L)]
                            * rj[slot][g * L + e0 + k, pl.ds(c * L, L)])
                for k in range(4):
                    red0[e0 + k, :] = accs[k]
            s16 = plsc.load_gather(red0, [lanes, jnp.full((L,), 0, jnp.int32)])
            for l in range(1, L):
                s16 = s16 + plsc.load_gather(
                    red0, [lanes, jnp.full((L,), l, jnp.int32)])
            s16 = jnp.where(s16 > 0.0, s16, ALPHA * s16)
            s_vv[slot, pl.ds(g * L, L)] = s16
            m_old = mz_v[0, :]
            m_new = jnp.maximum(m_old, s16)
            mz_v[1, :] = (mz_v[1, :] * jnp.exp(m_old - m_new)
                          + jnp.exp(s16 - m_new))
            mz_v[0, :] = m_new
        score_copy(slot, b).start()

    for c in idx_copies(0, 0):
        c.start()
    for c in idx_copies(0, 0):
        c.wait()
    for c in row_copies(0):
        c.start()
    for c in idx_copies(1, 1):
        c.start()

    @pl.loop(0, NBLK // 2)
    def _pipe(i):
        for par in range(2):
            b = 2 * i + par
            slot, other = par, 1 - par
            for c in row_copies(slot):
                c.wait()

            @pl.when(b < NBLK - 2)
            def _():
                for c in idx_copies(slot, b + 2):
                    c.start()
            for c in idx_copies(other, b + 1):
                c.wait()
            for c in row_copies(other):
                c.start()

            @pl.when(b >= 2)
            def _():
                score_copy(slot, b - 2).wait()
            compute(slot, b)

    bl = NBLK - 1
    for c in row_copies(bl % 2):
        c.wait()
    score_copy(bl % 2, bl - 2).wait()
    compute(bl % 2, bl)
    score_copy(1 - bl % 2, bl - 1).wait()
    score_copy(bl % 2, bl).wait()

    m_acc = mz_v[0, :]
    m_w = jnp.max(m_acc)
    mv = jnp.full((L,), m_w, jnp.float32)
    z_w = jnp.sum(mz_v[1, :] * jnp.exp(m_acc - mv))
    mz_v[0, :] = mv
    mz_v[1, :] = jnp.full((L,), z_w, jnp.float32)
    pltpu.sync_copy(mz_v, stats_hbm.at[wid])


@functools.partial(
    pl.kernel,
    out_type=jax.ShapeDtypeStruct((NC, N_NODES, D), jnp.float32),
    mesh=_mesh,
    scratch_types=[
        pltpu.VMEM((B, D), jnp.float32),
        pltpu.VMEM((B, D), jnp.float32),
        pltpu.VMEM((B, D), jnp.float32),
        pltpu.VMEM((6, B), jnp.int32),
        pltpu.VMEM((6, B), jnp.int32),
        pltpu.VMEM((2, B), jnp.float32),
        pltpu.VMEM((NW, 2, L), jnp.float32),
        pltpu.VMEM_SHARED((N_NODES, D), jnp.float32),
        pltpu.SemaphoreType.DMA,
        pltpu.SemaphoreType.DMA,
        pltpu.SemaphoreType.DMA,
        pltpu.SemaphoreType.DMA,
        pltpu.SemaphoreType.DMA,
        pltpu.SemaphoreType.DMA,
        pltpu.SemaphoreType.DMA,
        pltpu.SemaphoreType.DMA,
        pltpu.SemaphoreType.DMA,
        pltpu.SemaphoreType.DMA,
        pltpu.SemaphoreType.DMA,
        pltpu.SemaphoreType.DMA,
        pltpu.SemaphoreType.DMA,
        pltpu.SemaphoreType.DMA,
    ],
    compiler_params=_cp,
)
def _agg_kernel(emb_hbm, src2_hbm, dst2_hbm, scores_hbm, stats_hbm, part_hbm,
                rb0, rb1, rb2, src_ix, dst_ix, s_vv, st_v, acc_sh,
                bi0, bi1, bi2, bi3, bi4, bi5, br0, br1, br2,
                bw0, bw1, bw2, bs0, bs1):
    cid = lax.axis_index("c")
    sid = lax.axis_index("s")
    wid = cid * NS + sid
    base = wid * EDGES_PER_W
    wblk = wid * NBLK
    rb = (rb0, rb1, rb2)
    semi = (bi0, bi1, bi2, bi3, bi4, bi5)
    semr = (br0, br1, br2)
    semw = (bw0, bw1, bw2)
    sems = (bs0, bs1)

    pltpu.sync_copy(stats_hbm, st_v)
    m_v = st_v[0, 0, :]
    for w in range(1, NW):
        m_v = jnp.maximum(m_v, st_v[w, 0, :])
    z_all = st_v[0, 1, :] * jnp.exp(st_v[0, 0, :] - m_v)
    for w in range(1, NW):
        z_all = z_all + st_v[w, 1, :] * jnp.exp(st_v[w, 0, :] - m_v)
    inv_z = 1.0 / z_all

    @pl.loop(0, ZCH)
    def _zr(r):
        for c in range(D // L):
            rb0[r, pl.ds(c * L, L)] = jnp.zeros((L,), jnp.float32)

    @pl.loop(0, NZCH // NS)
    def _zc(i):
        pltpu.sync_copy(rb0, acc_sh.at[pl.ds((i * NS + sid) * ZCH, ZCH)])
    rem = NZCH % NS
    if rem:
        @pl.when(sid < rem)
        def _():
            pltpu.sync_copy(
                rb0, acc_sh.at[pl.ds(((NZCH - rem) + sid) * ZCH, ZCH)])
    plsc.subcore_barrier()

    def idx_copies(islot, b):
        return (pltpu.make_async_copy(src2_hbm.at[wblk + b],
                                      src_ix.at[islot], semi[islot]),
                pltpu.make_async_copy(dst2_hbm.at[wblk + b],
                                      dst_ix.at[islot], semi[islot]))

    def row_copy(rslot, islot):
        return pltpu.make_async_copy(emb_hbm.at[dst_ix.at[islot]],
                                     rb[rslot], semr[rslot])

    def score_copy(rslot, b):
        return pltpu.make_async_copy(
            scores_hbm.at[pl.ds(base + b * B, B)], s_vv.at[rslot], sems[rslot])

    class _Scatter:
        def __init__(self, rslot, islot):
            self._d = pltpu.make_async_copy(
                rb[rslot], acc_sh.at[src_ix.at[islot]], semw[rslot])

        def start(self):
            self._d.start(add=True)

        def wait(self):
            self._d.wait()

    def scatter_copy(rslot, islot):
        return _Scatter(rslot, islot)

    def compute(rslot, sslot):
        slot_v = jnp.full((L,), sslot, jnp.int32)

        @pl.loop(0, B, step=4)
        def _edge(e):
            wrows = []
            for k in range(4):
                srow = plsc.load_gather(
                    s_vv, [slot_v, jnp.full((L,), e + k, jnp.int32)])
                wrows.append(jnp.exp(srow - m_v) * inv_z)
            for c in range(D // L):
                for k in range(4):
                    rb[rslot][e + k, pl.ds(c * L, L)] = (
                        rb[rslot][e + k, pl.ds(c * L, L)] * wrows[k])

    for c in idx_copies(0, 0):
        c.start()
    for c in idx_copies(0, 0):
        c.wait()
    row_copy(0, 0).start()
    score_copy(0, 0).start()
    for c in idx_copies(1, 1):
        c.start()
    score_copy(1, 1).start()

    @pl.loop(0, (NBLK + 5) // 6)
    def _pipe(i):
        for par in range(6):
            b = 6 * i + par
            r3, r3n = par % 3, (par + 1) % 3
            s2 = par % 2

            @pl.when(b < NBLK)
            def _():
                row_copy(r3, par).wait()

            @pl.when((b >= 2) & (b < NBLK + 2))
            def _():
                scatter_copy((par + 1) % 3, (par + 4) % 6).wait()

            @pl.when(b < NBLK - 2)
            def _():
                for c in idx_copies((par + 2) % 6, b + 2):
                    c.start()

            @pl.when(b < NBLK - 1)
            def _():
                for c in idx_copies((par + 1) % 6, b + 1):
                    c.wait()
                row_copy(r3n, (par + 1) % 6).start()

            @pl.when(b < NBLK)
            def _():
                score_copy(s2, b).wait()
                compute(r3, s2)
                scatter_copy(r3, par).start()

            @pl.when(b < NBLK - 2)
            def _():
                score_copy(s2, b + 2).start()

    scatter_copy((NBLK - 1) % 3, (NBLK - 1) % 6).wait()
    plsc.subcore_barrier()

    for ch in range(NRCH):
        @pl.when(sid == (ch % NS))
        def _():
            pltpu.sync_copy(acc_sh.at[pl.ds(ch * RCH, RCH)],
                            part_hbm.at[cid, pl.ds(ch * RCH, RCH)])


def _final_body(e_ref, p0_ref, p1_ref, o_ref):
    o_ref[...] = e_ref[...] + p0_ref[...] + p1_ref[...]


def _final_add(emb, p0, p1):
    blk = 1000
    grid = N_NODES // blk
    spec = pl.BlockSpec((blk, D), lambda i: (i, 0))
    return pl.pallas_call(
        _final_body,
        out_shape=jax.ShapeDtypeStruct((N_NODES, D), jnp.float32),
        grid=(grid,),
        in_specs=[spec, spec, spec],
        out_specs=spec,
    )(emb, p0, p1)


def kernel(drug_embeddings, drug_relationships):
    src = drug_relationships[:, 0].astype(jnp.int32)
    dst = drug_relationships[:, 1].astype(jnp.int32)
    src2 = src.reshape(NW * NBLK, B)
    dst2 = dst.reshape(NW * NBLK, B)
    scores, stats = _score_kernel(drug_embeddings, src2, dst2)
    partial = _agg_kernel(drug_embeddings, src2, dst2, scores, stats)
    return _final_add(drug_embeddings, partial[0], partial[1])
